# baseline (speedup 1.0000x reference)
import jax
import jax.numpy as jnp
from jax import lax
from jax.experimental import pallas as pl
from jax.experimental.pallas import tpu as pltpu
from jax.experimental.pallas import tpu_sc as plsc

B = 16384
D = 16
NC = 2
NS = 16
NW = NC * NS
CHUNK = B // NW
G = 128
NGRP = CHUNK // G
SLAB = 128


def _gmf_body(user_hbm, item_hbm, utab_hbm, itab_hbm, wb_hbm, out_hbm,
              uidx_v, iidx_v, uhi_v, ihi_v,
              ubuf0, ubuf1, ibuf0, ibuf1, out_v, wb_v,
              sem_u0, sem_u1, sem_i0, sem_i1):
    wid = lax.axis_index("s") * NC + lax.axis_index("c")
    base = wid * CHUNK

    pltpu.sync_copy(wb_hbm, wb_v)
    pltpu.sync_copy(user_hbm.at[pl.ds(base, CHUNK)], uidx_v)
    pltpu.sync_copy(item_hbm.at[pl.ds(base, CHUNK)], iidx_v)

    def mk_hi(i, carry):
        sl = pl.ds(i * D, D)
        uhi_v[sl] = lax.shift_right_logical(uidx_v[sl], 3)
        ihi_v[sl] = lax.shift_right_logical(iidx_v[sl], 3)
        return carry
    lax.fori_loop(0, CHUNK // D, mk_hi, 0)

    ubufs = (ubuf0, ubuf1)
    ibufs = (ibuf0, ibuf1)
    usems = (sem_u0, sem_u1)
    isems = (sem_i0, sem_i1)

    def fire(g):
        sl = pl.ds(g * G, G)
        cu = pltpu.async_copy(
            utab_hbm.at[uhi_v.at[sl]], ubufs[g % 2], usems[g % 2])
        ci = pltpu.async_copy(
            itab_hbm.at[ihi_v.at[sl]], ibufs[g % 2], isems[g % 2])
        return cu, ci

    bias_v = wb_v[pl.ds(D, D)]
    lane = lax.iota(jnp.int32, D)
    rot = [(lane + d) & (D - 1) for d in range(D)]
    wrot = [plsc.load_gather(wb_v, [rot[d]]) for d in range(D)]

    def compute_group(g):
        buf = g % 2
        for blk in range(G // D):
            rbase = g * G + blk * D
            uvec = uidx_v[pl.ds(rbase, D)]
            ivec = iidx_v[pl.ds(rbase, D)]
            rows = lane + blk * D
            uoff = (uvec & 7) * D
            ioff = (ivec & 7) * D
            acc = bias_v
            for d in range(D):
                uc = plsc.load_gather(ubufs[buf], [rows, uoff + rot[d]])
                ic = plsc.load_gather(ibufs[buf], [rows, ioff + rot[d]])
                acc = acc + uc * ic * wrot[d]
            out_v[pl.ds(rbase, D)] = jnp.maximum(acc, 0.0)

    prev = fire(0)
    for g in range(NGRP):
        nxt = fire(g + 1) if g + 1 < NGRP else None
        prev[0].wait()
        prev[1].wait()
        compute_group(g)
        prev = nxt

    pltpu.sync_copy(out_v, out_hbm.at[pl.ds(base, CHUNK)])


def kernel(user, item, user_table, item_table, W, b):
    u32 = user.astype(jnp.int32)
    i32_ = item.astype(jnp.int32)
    one = b[0] * 0.0 + 1.0
    ut = user_table.reshape(125000, SLAB) * one
    it = item_table.reshape(125000, SLAB) * one
    wb = jnp.concatenate([W.reshape(D), jnp.broadcast_to(b, (D,))])

    mesh = plsc.VectorSubcoreMesh(
        core_axis_name="c", subcore_axis_name="s",
        num_cores=NC, num_subcores=NS)

    run = pl.kernel(
        _gmf_body,
        out_type=jax.ShapeDtypeStruct((B,), jnp.float32),
        mesh=mesh,
        compiler_params=pltpu.CompilerParams(
            needs_layout_passes=False, use_tc_tiling_on_sc=False),
        scratch_types=[
            pltpu.VMEM((CHUNK,), jnp.int32),
            pltpu.VMEM((CHUNK,), jnp.int32),
            pltpu.VMEM((CHUNK,), jnp.int32),
            pltpu.VMEM((CHUNK,), jnp.int32),
            pltpu.VMEM((G, SLAB), jnp.float32),
            pltpu.VMEM((G, SLAB), jnp.float32),
            pltpu.VMEM((G, SLAB), jnp.float32),
            pltpu.VMEM((G, SLAB), jnp.float32),
            pltpu.VMEM((CHUNK,), jnp.float32),
            pltpu.VMEM((2 * D,), jnp.float32),
            pltpu.SemaphoreType.DMA,
            pltpu.SemaphoreType.DMA,
            pltpu.SemaphoreType.DMA,
            pltpu.SemaphoreType.DMA,
        ],
    )
    out = run(u32, i32_, ut, it, wb)
    return out.reshape(B, 1)


# speedup vs baseline: 1.5316x; 1.5316x over previous
import jax
import jax.numpy as jnp
from jax import lax
from jax.experimental import pallas as pl
from jax.experimental.pallas import tpu as pltpu
from jax.experimental.pallas import tpu_sc as plsc

B = 16384
D = 16
NC = 2
NS = 16
NW = NC * NS
CHUNK = B // NW
GSZ = 128
NG = CHUNK // GSZ
BLK = CHUNK // D


def _gmf_body(user_hbm, item_hbm, utab_hbm, itab_hbm, wb_hbm, out_hbm,
              uidx_v, iidx_v, urows_v, irows_v, out_v, wb_v, tbuf_v,
              sem_u, sem_i):
    wid = lax.axis_index("s") * NC + lax.axis_index("c")
    base = wid * CHUNK

    pltpu.sync_copy(wb_hbm, wb_v)
    pltpu.sync_copy(user_hbm.at[pl.ds(base, CHUNK)], uidx_v)
    pltpu.sync_copy(item_hbm.at[pl.ds(base, CHUNK)], iidx_v)

    copies = []
    for g in range(NG):
        sl = pl.ds(g * GSZ, GSZ)
        copies.append(pltpu.async_copy(
            utab_hbm.at[uidx_v.at[sl]], urows_v.at[sl], sem_u))
        copies.append(pltpu.async_copy(
            itab_hbm.at[iidx_v.at[sl]], irows_v.at[sl], sem_i))
    for c in copies:
        c.wait()

    wv = wb_v[pl.ds(0, D)]
    bias_v = wb_v[pl.ds(D, D)]
    col_idx = lax.iota(jnp.int32, D) * (D + 1)

    def block(blk, carry):
        rbase = blk * D
        for j in range(D):
            u = urows_v[rbase + j, :]
            i = irows_v[rbase + j, :]
            plsc.store_scatter(tbuf_v, [col_idx + j], u * i * wv)
        acc = bias_v
        for d in range(D):
            acc = acc + tbuf_v[pl.ds(d * (D + 1), D)]
        out_v[pl.ds(rbase, D)] = jnp.maximum(acc, 0.0)
        return carry

    lax.fori_loop(0, BLK, block, 0)
    pltpu.sync_copy(out_v, out_hbm.at[pl.ds(base, CHUNK)])


def kernel(user, item, user_table, item_table, W, b):
    user1 = user.astype(jnp.int32)
    item1 = item.astype(jnp.int32)
    wb = jnp.concatenate([W.reshape(D), jnp.broadcast_to(b, (D,))])

    mesh = plsc.VectorSubcoreMesh(
        core_axis_name="c", subcore_axis_name="s",
        num_cores=NC, num_subcores=NS)

    run = pl.kernel(
        _gmf_body,
        out_type=jax.ShapeDtypeStruct((B,), jnp.float32),
        mesh=mesh,
        compiler_params=pltpu.CompilerParams(
            needs_layout_passes=False, use_tc_tiling_on_sc=False),
        scratch_types=[
            pltpu.VMEM((CHUNK,), jnp.int32),
            pltpu.VMEM((CHUNK,), jnp.int32),
            pltpu.VMEM((CHUNK, D), jnp.float32),
            pltpu.VMEM((CHUNK, D), jnp.float32),
            pltpu.VMEM((CHUNK,), jnp.float32),
            pltpu.VMEM((2 * D,), jnp.float32),
            pltpu.VMEM((D * (D + 1),), jnp.float32),
            pltpu.SemaphoreType.DMA,
            pltpu.SemaphoreType.DMA,
        ],
    )
    out = run(user1, item1, user_table, item_table, wb)
    return out.reshape(B, 1)
